# baseline (speedup 1.0000x reference)
import functools
import math

import jax
import jax.numpy as jnp
from jax import lax
from jax.experimental import pallas as pl
from jax.experimental.pallas import tpu as pltpu
from jax.experimental.pallas import tpu_sc as plsc

D_MODEL = 64
SCALE = math.sqrt(D_MODEL)
NUM_CORES = 2
NUM_SUBCORES = 16
NUM_WORKERS = NUM_CORES * NUM_SUBCORES
LANES = 16
PAIR = 2 * D_MODEL


@functools.lru_cache(maxsize=None)
def _make_emb_kernel(NB: int, S: int, V2: int):
    chunk = S
    B = NB * S
    assert B % (NUM_WORKERS * chunk) == 0
    b_per_w = B // NUM_WORKERS
    nb_per_w = NB // NUM_WORKERS
    n_chunks = b_per_w // chunk
    assert n_chunks % 2 == 0
    ndma = -(-chunk // LANES)
    gchunk = ndma * LANES
    mesh = plsc.VectorSubcoreMesh(
        core_axis_name="c",
        subcore_axis_name="s",
        num_cores=NUM_CORES,
        num_subcores=NUM_SUBCORES,
    )

    @functools.partial(
        pl.kernel,
        out_type=jax.ShapeDtypeStruct((NB, S, D_MODEL), jnp.float32),
        mesh=mesh,
        scratch_types=[
            pltpu.VMEM((b_per_w + LANES,), jnp.int32),
            pltpu.VMEM((gchunk,), jnp.int32),
            pltpu.VMEM((gchunk,), jnp.int32),
            pltpu.VMEM((gchunk, PAIR), jnp.float32),
            pltpu.VMEM((gchunk, PAIR), jnp.float32),
            pltpu.VMEM((1, S, D_MODEL), jnp.float32),
            pltpu.VMEM((1, S, D_MODEL), jnp.float32),
            pltpu.SemaphoreType.DMA,
            pltpu.SemaphoreType.DMA,
            pltpu.SemaphoreType.DMA,
            pltpu.SemaphoreType.DMA,
        ],
        compiler_params=pltpu.CompilerParams(
            use_tc_tiling_on_sc=True,
            disable_bounds_checks=True,
            skip_device_barrier=True,
        ),
    )
    def emb(x_hbm, table_hbm, out_hbm, idx_all, i0, i1, g0, g1, s0, s1,
            gsem0, gsem1, osem0, osem1):
        wid = lax.axis_index("s") * NUM_CORES + lax.axis_index("c")
        base = wid * b_per_w
        nb_base = wid * nb_per_w
        ibufs, gbufs, sbufs = (i0, i1), (g0, g1), (s0, s1)
        gsems, osems = (gsem0, gsem1), (osem0, osem1)

        pltpu.sync_copy(x_hbm.at[pl.ds(base, b_per_w)],
                        idx_all.at[pl.ds(0, b_per_w)])
        idx_all[pl.ds(b_per_w, LANES)] = jnp.zeros((LANES,), jnp.int32)

        def make_pair_idx(k, b):
            ib = ibufs[b]

            def vb(i, c):
                sl = pl.ds(i * LANES, LANES)
                ib[sl] = lax.shift_right_logical(
                    idx_all[pl.ds(k * chunk + i * LANES, LANES)], 1)
                return c

            lax.fori_loop(0, ndma, vb, 0, unroll=4)

        def gstart(b):
            for i in range(ndma):
                iv = ibufs[b][pl.ds(i * LANES, LANES)]
                pltpu.async_copy(
                    table_hbm.at[iv],
                    gbufs[b].at[pl.ds(i * LANES, LANES)], gsems[b])

        def gwait(b):
            pltpu.make_async_copy(
                table_hbm.at[ibufs[b]], gbufs[b], gsems[b]).wait()

        def scopy(k, b):
            return pltpu.make_async_copy(
                sbufs[b], out_hbm.at[pl.ds(nb_base + k, 1)], osems[b])

        def scale(k, b):
            gb, sb = gbufs[b], sbufs[b]

            def row_body(r, c):
                iv = idx_all[pl.ds(k * chunk + r, LANES)]
                off = (iv[0] & 1) * D_MODEL
                for j in range(D_MODEL // LANES):
                    sb[0, r, pl.ds(j * LANES, LANES)] = (
                        gb[r, pl.ds(off + j * LANES, LANES)] * SCALE)
                return c

            lax.fori_loop(0, chunk, row_body, 0, unroll=4)

        make_pair_idx(0, 0)
        gstart(0)
        make_pair_idx(1, 1)
        gstart(1)

        def pair_body(h, carry):
            for b in range(2):
                k = 2 * h + b
                gwait(b)

                @pl.when(k >= 2)
                def _():
                    scopy(k - 2, b).wait()

                scale(k, b)
                scopy(k, b).start()

                @pl.when(k + 2 < n_chunks)
                def _():
                    make_pair_idx(k + 2, b)
                    gstart(b)
            return carry

        lax.fori_loop(0, n_chunks // 2, pair_body, 0)
        for b in range(2):
            scopy(n_chunks - 2 + b, b).wait()

    return emb


def kernel(x, table):
    NB, S = x.shape
    V = table.shape[0]
    xf = x.reshape(NB * S).astype(jnp.int32)
    tv = table.reshape(V // 2, PAIR)
    return _make_emb_kernel(NB, S, V // 2)(xf, tv)


# speedup vs baseline: 1.3269x; 1.3269x over previous
import functools
import math

import jax
import jax.numpy as jnp
from jax import lax
from jax.experimental import pallas as pl
from jax.experimental.pallas import tpu as pltpu
from jax.experimental.pallas import tpu_sc as plsc

D_MODEL = 64
SCALE = math.sqrt(D_MODEL)
NUM_CORES = 2
NUM_SUBCORES = 16
NUM_WORKERS = NUM_CORES * NUM_SUBCORES
CHUNK = 512
LANES = 16


@functools.lru_cache(maxsize=None)
def _make_emb_kernel(B: int, V: int):
    assert B % (NUM_WORKERS * CHUNK) == 0
    b_per_w = B // NUM_WORKERS
    n_chunks = b_per_w // CHUNK
    assert n_chunks % 2 == 0
    mesh = plsc.VectorSubcoreMesh(
        core_axis_name="c",
        subcore_axis_name="s",
        num_cores=NUM_CORES,
        num_subcores=NUM_SUBCORES,
    )

    @functools.partial(
        pl.kernel,
        out_type=jax.ShapeDtypeStruct((B, D_MODEL), jnp.float32),
        mesh=mesh,
        scratch_types=[
            pltpu.VMEM((CHUNK,), jnp.int32),
            pltpu.VMEM((CHUNK,), jnp.int32),
            pltpu.VMEM((CHUNK, D_MODEL), jnp.float32),
            pltpu.VMEM((CHUNK, D_MODEL), jnp.float32),
            pltpu.SemaphoreType.DMA,
            pltpu.SemaphoreType.DMA,
            pltpu.SemaphoreType.DMA,
            pltpu.SemaphoreType.DMA,
        ],
        compiler_params=pltpu.CompilerParams(use_tc_tiling_on_sc=False),
    )
    def emb(x_hbm, table_hbm, out_hbm, i0, i1, r0, r1,
            gsem0, gsem1, osem0, osem1):
        wid = lax.axis_index("s") * NUM_CORES + lax.axis_index("c")
        base = wid * b_per_w
        ibufs, rbufs = (i0, i1), (r0, r1)
        gsems, osems = (gsem0, gsem1), (osem0, osem1)

        def scopy(k, b):
            return pltpu.make_async_copy(
                rbufs[b], out_hbm.at[pl.ds(base + k * CHUNK, CHUNK)],
                osems[b])

        def chunk_work(k, b):
            off = base + k * CHUNK
            pltpu.sync_copy(x_hbm.at[pl.ds(off, CHUNK)], ibufs[b])
            pltpu.async_copy(table_hbm.at[ibufs[b]], rbufs[b],
                             gsems[b]).wait()

            def row_body(r, c):
                for j in range(D_MODEL // LANES):
                    sl = pl.ds(j * LANES, LANES)
                    rbufs[b][r, sl] = rbufs[b][r, sl] * SCALE
                return c

            lax.fori_loop(0, CHUNK, row_body, 0, unroll=4)
            scopy(k, b).start()

        def pair_body(h, carry):
            for b in range(2):
                k = 2 * h + b

                @pl.when(k >= 2)
                def _():
                    scopy(k - 2, b).wait()

                chunk_work(k, b)
            return carry

        lax.fori_loop(0, n_chunks // 2, pair_body, 0)
        for b in range(2):
            scopy(n_chunks - 2 + b, b).wait()

    return emb


def kernel(x, table):
    B = x.size
    xf = x.reshape(B).astype(jnp.int32)
    out = _make_emb_kernel(B, table.shape[0])(xf, table)
    return out.reshape(*x.shape, D_MODEL)
